# baseline (speedup 1.0000x reference)
import jax
import jax.numpy as jnp
from jax.experimental import pallas as pl

NCLS = 3
NBINS = 16384


def kernel(inputs, targets):
    probas = jax.nn.softmax(inputs, axis=1)
    vp = jnp.transpose(probas, (0, 2, 3, 1)).reshape(-1, NCLS)
    vt = targets.reshape(-1)
    total = jnp.float32(0.0)
    count = jnp.float32(0.0)
    for c in range(NCLS):
        fg = (vt == c).astype(jnp.float32)
        e = jnp.abs(fg - vp[:, c])
        binidx = jnp.clip((e * NBINS).astype(jnp.int32), 0, NBINS - 1)
        n_b = jnp.zeros(NBINS, jnp.float32).at[binidx].add(1.0)
        m_b = jnp.zeros(NBINS, jnp.float32).at[binidx].add(fg)
        rn = jnp.cumsum(n_b[::-1])[::-1]
        rm = jnp.cumsum(m_b[::-1])[::-1]
        S = rm[0]
        denom = S + rn - rm
        J = jnp.where(rn > 0, rn / jnp.where(denom > 0, denom, 1.0), 0.0)
        Jn = jnp.concatenate([J[1:], jnp.zeros(1, jnp.float32)])
        v = (jnp.arange(NBINS, dtype=jnp.float32) + 0.5) / NBINS
        loss_c = jnp.sum(v * (J - Jn))
        include = jnp.logical_or(S != 0.0, c == 0)
        total = total + jnp.where(include, loss_c, jnp.float32(0.0))
        count = count + jnp.where(include, jnp.float32(1.0), jnp.float32(0.0))
    return total / jnp.maximum(count, jnp.float32(1.0))


# speedup vs baseline: 58.7611x; 58.7611x over previous
import functools

import jax
import jax.numpy as jnp
from jax import lax
from jax.experimental import pallas as pl
from jax.experimental.pallas import tpu as pltpu
from jax.experimental.pallas import tpu_sc as plsc

NCLS = 3
NBINS = 8192
NPIX_B = 512 * 512
NBATCH = 16
NPIX = NBATCH * NPIX_B
NWORK = 32
PIX_W = NPIX // NWORK
CH = 16384
NCH = PIX_W // CH
LANES = 16


def _sc_hist_kernel(x_hbm, t_hbm, out_hbm, x0, x1, x2, lb,
                    h0n, h0m, h1n, h1m, h2n, h2m):
    wid = lax.axis_index("s") * 2 + lax.axis_index("c")
    b = wid // 2
    half = wid % 2
    hists = (h0n, h0m, h1n, h1m, h2n, h2m)

    def zero_body(i, carry):
        z = jnp.zeros((LANES,), jnp.float32)
        for hr in hists:
            hr[pl.ds(i * LANES, LANES)] = z
        return carry

    lax.fori_loop(0, NBINS // LANES, zero_body, 0)

    base_pix = half * PIX_W
    ones = jnp.ones((LANES,), jnp.float32)
    zeros = jnp.zeros((LANES,), jnp.float32)
    scale = jnp.full((LANES,), float(NBINS), jnp.float32)
    top = jnp.full((LANES,), NBINS - 1, jnp.int32)
    bot = jnp.zeros((LANES,), jnp.int32)

    for k in range(NCH):
        off = base_pix + k * CH
        pltpu.sync_copy(t_hbm.at[pl.ds(b * NPIX_B + off, CH)], lb)
        for c, buf in ((0, x0), (1, x1), (2, x2)):
            pltpu.sync_copy(
                x_hbm.at[pl.ds((b * NCLS + c) * NPIX_B + off, CH)], buf)

        def body(i, carry):
            s = i * LANES
            v0 = x0[pl.ds(s, LANES)]
            v1 = x1[pl.ds(s, LANES)]
            v2 = x2[pl.ds(s, LANES)]
            t = lb[pl.ds(s, LANES)]
            e0 = jnp.exp(v0)
            e1 = jnp.exp(v1)
            e2 = jnp.exp(v2)
            zsum = e0 + e1 + e2
            r = scale / zsum
            for c, (ec, hn, hm) in enumerate(((e0, h0n, h0m),
                                              (e1, h1n, h1m),
                                              (e2, h2n, h2m))):
                fgm = t == c
                num = jnp.where(fgm, zsum - ec, ec)
                bidx = (num * r).astype(jnp.int32)
                bidx = jnp.minimum(jnp.maximum(bidx, bot), top)
                plsc.addupdate_scatter(hn, [bidx], ones)
                plsc.addupdate_scatter(hm, [bidx],
                                       jnp.where(fgm, ones, zeros))
            return carry

        lax.fori_loop(0, CH // LANES, body, 0)

    for j, hr in enumerate(hists):
        pltpu.sync_copy(hr, out_hbm.at[wid, j])


def _tc_scan_kernel(hist_ref, out_ref):
    hs = jnp.sum(hist_ref[...], axis=0)
    rows = NBINS // 128
    ri = lax.broadcasted_iota(jnp.int32, (128, 128), 0)
    ci = lax.broadcasted_iota(jnp.int32, (128, 128), 1)
    tri = (ri <= ci).astype(jnp.float32)
    rl = lax.broadcasted_iota(jnp.int32, (rows, rows), 0)
    cl = lax.broadcasted_iota(jnp.int32, (rows, rows), 1)
    low = (cl < rl).astype(jnp.float32)

    total = jnp.zeros((1, 1), jnp.float32)
    count = jnp.zeros((1, 1), jnp.float32)
    for c in range(NCLS):
        n = hs[2 * c].reshape(rows, 128)
        m = hs[2 * c + 1].reshape(rows, 128)
        csn = jnp.dot(n, tri, precision=lax.Precision.HIGHEST)
        csm = jnp.dot(m, tri, precision=lax.Precision.HIGHEST)
        offn = jnp.dot(low, csn[:, 127:128], precision=lax.Precision.HIGHEST)
        offm = jnp.dot(low, csm[:, 127:128], precision=lax.Precision.HIGHEST)
        csn = csn + offn
        csm = csm + offm
        ntot = jnp.sum(n)
        stot = jnp.sum(m)
        rn = ntot - csn + n
        rm = stot - csm + m
        denom = stot + rn - rm
        j_b = jnp.where(rn > 0.0, rn / jnp.where(denom > 0.0, denom, 1.0),
                        0.0)
        loss_c = (jnp.sum(j_b) - 0.5) / float(NBINS)
        inc = jnp.where(jnp.logical_or(stot > 0.0, c == 0), 1.0, 0.0)
        total = total + loss_c * inc
        count = count + inc
    out_ref[...] = total / jnp.maximum(count, 1.0)


_sc_hist = functools.partial(
    pl.kernel,
    mesh=plsc.VectorSubcoreMesh(core_axis_name="c", subcore_axis_name="s"),
    out_type=jax.ShapeDtypeStruct((NWORK, 2 * NCLS, NBINS), jnp.float32),
    compiler_params=pltpu.CompilerParams(needs_layout_passes=False),
    scratch_types=[
        pltpu.VMEM((CH,), jnp.float32),
        pltpu.VMEM((CH,), jnp.float32),
        pltpu.VMEM((CH,), jnp.float32),
        pltpu.VMEM((CH,), jnp.int32),
        pltpu.VMEM((NBINS,), jnp.float32),
        pltpu.VMEM((NBINS,), jnp.float32),
        pltpu.VMEM((NBINS,), jnp.float32),
        pltpu.VMEM((NBINS,), jnp.float32),
        pltpu.VMEM((NBINS,), jnp.float32),
        pltpu.VMEM((NBINS,), jnp.float32),
    ],
)(_sc_hist_kernel)


_tc_scan = pl.pallas_call(
    _tc_scan_kernel,
    out_shape=jax.ShapeDtypeStruct((1, 1), jnp.float32),
)


def kernel(inputs, targets):
    x_flat = inputs.reshape(-1)
    t_flat = targets.reshape(-1)
    hists = _sc_hist(x_flat, t_flat)
    out = _tc_scan(hists)
    return out[0, 0]


# speedup vs baseline: 69.0055x; 1.1743x over previous
import functools

import jax
import jax.numpy as jnp
from jax import lax
from jax.experimental import pallas as pl
from jax.experimental.pallas import tpu as pltpu
from jax.experimental.pallas import tpu_sc as plsc

NCLS = 3
NBINS = 8192
H = 512
W = 512
NBATCH = 16
NWORK = 32
ROWS_W = NBATCH * H // NWORK
CHR = 16
NCH = ROWS_W // CHR
LANES = 16
VECS = CHR * W // LANES
VPR = W // LANES


def _sc_hist_kernel(x_hbm, t_hbm, out_hbm, xb0, xb1, xb2, lbb, h0, h1, h2):
    wid = lax.axis_index("s") * 2 + lax.axis_index("c")
    b = wid // 2
    half = wid % 2
    hists = (h0, h1, h2)

    def zero_body(i, carry):
        z = jnp.zeros((LANES,), jnp.float32)
        for hr in hists:
            hr[pl.ds(i * LANES, LANES)] = z
        return carry

    lax.fori_loop(0, 2 * NBINS // LANES, zero_body, 0)

    row0 = half * ROWS_W
    ones = jnp.ones((LANES,), jnp.float32)
    scale = jnp.full((LANES,), float(NBINS), jnp.float32)
    top = jnp.full((LANES,), NBINS - 1, jnp.int32)
    bot = jnp.zeros((LANES,), jnp.int32)
    fgoff = jnp.full((LANES,), NBINS, jnp.int32)
    izero = jnp.zeros((LANES,), jnp.int32)

    def chunk_body(k, carry):
        r0 = row0 + k * CHR
        pltpu.sync_copy(t_hbm.at[pl.ds(b, 1), pl.ds(r0, CHR), :], lbb)
        pltpu.sync_copy(x_hbm.at[pl.ds(b, 1), pl.ds(0, 1), pl.ds(r0, CHR), :],
                        xb0)
        pltpu.sync_copy(x_hbm.at[pl.ds(b, 1), pl.ds(1, 1), pl.ds(r0, CHR), :],
                        xb1)
        pltpu.sync_copy(x_hbm.at[pl.ds(b, 1), pl.ds(2, 1), pl.ds(r0, CHR), :],
                        xb2)

        def body(i, c2):
            r = i // VPR
            col = (i % VPR) * LANES
            v0 = xb0[0, 0, r, pl.ds(col, LANES)]
            v1 = xb1[0, 0, r, pl.ds(col, LANES)]
            v2 = xb2[0, 0, r, pl.ds(col, LANES)]
            t = lbb[0, r, pl.ds(col, LANES)]
            e0 = jnp.exp(v0)
            e1 = jnp.exp(v1)
            e2 = jnp.exp(v2)
            zsum = e0 + e1 + e2
            rs = scale / zsum
            for c, (ec, hr) in enumerate(((e0, h0), (e1, h1), (e2, h2))):
                fgm = t == c
                num = jnp.where(fgm, zsum - ec, ec)
                bidx = (num * rs).astype(jnp.int32)
                bidx = jnp.minimum(jnp.maximum(bidx, bot), top)
                bidx = bidx + jnp.where(fgm, fgoff, izero)
                plsc.addupdate_scatter(hr, [bidx], ones)
            return c2

        lax.fori_loop(0, VECS, body, 0)
        return carry

    lax.fori_loop(0, NCH, chunk_body, 0)

    for c, hr in enumerate(hists):
        pltpu.sync_copy(hr.at[pl.ds(0, NBINS)], out_hbm.at[wid, 2 * c])
        pltpu.sync_copy(hr.at[pl.ds(NBINS, NBINS)],
                        out_hbm.at[wid, 2 * c + 1])


def _tc_scan_kernel(hist_ref, out_ref):
    hs = jnp.sum(hist_ref[...], axis=0)
    rows = NBINS // 128
    ri = lax.broadcasted_iota(jnp.int32, (128, 128), 0)
    ci = lax.broadcasted_iota(jnp.int32, (128, 128), 1)
    tri = (ri <= ci).astype(jnp.float32)
    rl = lax.broadcasted_iota(jnp.int32, (rows, rows), 0)
    cl = lax.broadcasted_iota(jnp.int32, (rows, rows), 1)
    low = (cl < rl).astype(jnp.float32)

    total = jnp.zeros((1, 1), jnp.float32)
    count = jnp.zeros((1, 1), jnp.float32)
    for c in range(NCLS):
        m = hs[2 * c + 1].reshape(rows, 128)
        n = hs[2 * c].reshape(rows, 128) + m
        csn = jnp.dot(n, tri, precision=lax.Precision.HIGHEST)
        csm = jnp.dot(m, tri, precision=lax.Precision.HIGHEST)
        offn = jnp.dot(low, csn[:, 127:128], precision=lax.Precision.HIGHEST)
        offm = jnp.dot(low, csm[:, 127:128], precision=lax.Precision.HIGHEST)
        csn = csn + offn
        csm = csm + offm
        ntot = jnp.sum(n)
        stot = jnp.sum(m)
        rn = ntot - csn + n
        rm = stot - csm + m
        denom = stot + rn - rm
        j_b = jnp.where(rn > 0.0, rn / jnp.where(denom > 0.0, denom, 1.0),
                        0.0)
        loss_c = (jnp.sum(j_b) - 0.5) / float(NBINS)
        inc = jnp.where(jnp.logical_or(stot > 0.0, c == 0), 1.0, 0.0)
        total = total + loss_c * inc
        count = count + inc
    out_ref[...] = total / jnp.maximum(count, 1.0)


_sc_hist = functools.partial(
    pl.kernel,
    mesh=plsc.VectorSubcoreMesh(core_axis_name="c", subcore_axis_name="s"),
    out_type=jax.ShapeDtypeStruct((NWORK, 2 * NCLS, NBINS), jnp.float32),
    compiler_params=pltpu.CompilerParams(needs_layout_passes=False),
    scratch_types=[
        pltpu.VMEM((1, 1, CHR, W), jnp.float32),
        pltpu.VMEM((1, 1, CHR, W), jnp.float32),
        pltpu.VMEM((1, 1, CHR, W), jnp.float32),
        pltpu.VMEM((1, CHR, W), jnp.int32),
        pltpu.VMEM((2 * NBINS,), jnp.float32),
        pltpu.VMEM((2 * NBINS,), jnp.float32),
        pltpu.VMEM((2 * NBINS,), jnp.float32),
    ],
)(_sc_hist_kernel)


_tc_scan = pl.pallas_call(
    _tc_scan_kernel,
    out_shape=jax.ShapeDtypeStruct((1, 1), jnp.float32),
)


def kernel(inputs, targets):
    hists = _sc_hist(inputs, targets)
    out = _tc_scan(hists)
    return out[0, 0]
